# baseline (speedup 1.0000x reference)
import functools

import jax
import jax.numpy as jnp
from jax import lax
from jax.experimental import pallas as pl
from jax.experimental.pallas import tpu as pltpu
from jax.experimental.pallas import tpu_sc as plsc

H = 128
L = 3
ROW_BLK = 512

N_DRUG, N_PROT, N_EFF = 10000, 50000, 20000
E_DP, E_DD, E_PP, E_DE = 200000, 100000, 200000, 100000

CHUNK = 3712
NDUMP = 64
GB = 256

SENTINEL = 2147483647


def _round_up(x, m):
    return (x + m - 1) // m * m


def _ept(e):
    return _round_up((e + 15) // 16, 512)


EPT_DP, EPT_DD, EPT_PP, EPT_DE = _ept(E_DP), _ept(E_DD), _ept(E_PP), _ept(E_DE)
EPT_MAX = max(EPT_DP, EPT_DD, EPT_PP, EPT_DE)

CP_DD = _round_up(N_DRUG // 16 + 1, 128)
CP_DP = _round_up(N_PROT // 16 + 1, 128)
CP_DE = _round_up(N_EFF // 16 + 1, 128)
CL_DD, CL_DP, CL_DE = 16 * CP_DD, 16 * CP_DP, 16 * CP_DE
CP_MAX = CP_DP
CL_MAX = CL_DP


SEG = CL_MAX
ZB = 1600


def _counts_body(ddd, ddp, dpp, dde,
                 o_dd, o_dp, o_pp, o_de,
                 raw_d, stg, ones_v, zbuf, csp):
    sc = lax.axis_index("c")
    t = lax.axis_index("s")
    iota = lax.iota(jnp.int32, 16)
    one16 = jnp.ones((16,), jnp.float32)
    zero16 = jnp.zeros((16,), jnp.float32)

    for k in range(8):
        ones_v[pl.ds(k * 16, 16)] = one16

    def z(i, _):
        zbuf[pl.ds(i * 16, 16)] = zero16
        return 0
    lax.fori_loop(0, ZB // 16, z, 0)
    for i in range(4):
        pltpu.sync_copy(zbuf.at[pl.ds(0, ZB)],
                        csp.at[pl.ds(t * 4 * ZB + i * ZB, ZB)])
    plsc.subcore_barrier()

    def do_rel(dsts, ept, n_dst, segbase):
        base = t * ept
        pltpu.sync_copy(dsts.at[pl.ds(base, ept)], raw_d.at[pl.ds(0, ept)])

        def b_loop(b, _):
            for k in range(8):
                dv = raw_d[pl.ds(b * 128 + k * 16, 16)]
                dv = jnp.where(dv < n_dst, dv, n_dst + iota) + segbase
                stg[0, pl.ds(k * 16, 16)] = dv
            pltpu.sync_copy(ones_v, csp.at[stg.at[0]], add=True)
            return 0
        lax.fori_loop(0, ept // 128, b_loop, 0)

    @pl.when(sc == 0)
    def _():
        do_rel(ddd, EPT_DD, N_DRUG, 0)
        do_rel(ddp, EPT_DP, N_PROT, SEG)

    @pl.when(sc == 1)
    def _():
        do_rel(dpp, EPT_PP, N_PROT, 0)
        do_rel(dde, EPT_DE, N_EFF, SEG)

    plsc.subcore_barrier()

    @pl.when(sc == 0)
    def _():
        pltpu.sync_copy(csp.at[pl.ds(t * CP_DD, CP_DD)],
                        o_dd.at[pl.ds(t * CP_DD, CP_DD)])
        pltpu.sync_copy(csp.at[pl.ds(SEG + t * CP_DP, CP_DP)],
                        o_dp.at[pl.ds(t * CP_DP, CP_DP)])

    @pl.when(sc == 1)
    def _():
        pltpu.sync_copy(csp.at[pl.ds(t * CP_DP, CP_DP)],
                        o_pp.at[pl.ds(t * CP_DP, CP_DP)])
        pltpu.sync_copy(csp.at[pl.ds(SEG + t * CP_DE, CP_DE)],
                        o_de.at[pl.ds(t * CP_DE, CP_DE)])


def _counts(ddd, ddp, dpp, dde):
    mesh = plsc.VectorSubcoreMesh(core_axis_name="c", subcore_axis_name="s")
    f = pl.kernel(
        _counts_body,
        out_type=[jax.ShapeDtypeStruct((CL_DD,), jnp.float32),
                  jax.ShapeDtypeStruct((CL_DP,), jnp.float32),
                  jax.ShapeDtypeStruct((CL_DP,), jnp.float32),
                  jax.ShapeDtypeStruct((CL_DE,), jnp.float32)],
        mesh=mesh,
        scratch_types=[
            pltpu.VMEM((EPT_MAX,), jnp.int32),
            pltpu.VMEM((1, 128), jnp.int32),
            pltpu.VMEM((128,), jnp.float32),
            pltpu.VMEM((ZB,), jnp.float32),
            pltpu.VMEM_SHARED((2 * CL_MAX,), jnp.float32),
        ],
    )
    return f(ddd, ddp, dpp, dde)


REL_DD = dict(nch=3, k0=2, ept=EPT_DD)
REL_DP = dict(nch=14, k0=7, ept=EPT_DP)
REL_PP = dict(nch=14, k0=7, ept=EPT_PP)
REL_DE = dict(nch=6, k0=3, ept=EPT_DE)

CHUNK_PT = CHUNK // 16
DR_SPANS = [(0, 64), (64, 64), (128, 64), (192, 40)]

SHIFT = 14
DMASK = (1 << SHIFT) - 1


def _bk_len(rel):
    return 256 * (rel["nch"] + 1) * (rel["ept"] // 16)


def _cnt_len(rel):
    return 16 * (rel["nch"] + 1) * 16


def _prep_body(sdd, ddd, sdp, ddp, spp, dpp, sde, dde,
               bs_dd, bd_dd, cnt_dd, bs_dp, bd_dp, cnt_dp,
               bs_pp, bd_pp, cnt_pp, bs_de, bd_de, cnt_de,
               raw_s, raw_d, svbuf, dlbuf, posbuf, cntbuf, semp):
    sc = lax.axis_index("c")
    t = lax.axis_index("s")
    iota = lax.iota(jnp.int32, 16)

    def do_rel(srcs, dsts, bks, bkd, cnt, rel, n_dst):
        ept, nch = rel["ept"], rel["nch"]
        capl = ept // 16
        base = t * ept
        pltpu.sync_copy(srcs.at[pl.ds(base, ept)], raw_s.at[pl.ds(0, ept)])
        pltpu.sync_copy(dsts.at[pl.ds(base, ept)], raw_d.at[pl.ds(0, ept)])
        lane_base = (t * 16 + iota) * (nch + 1)
        zero16i = jnp.zeros((16,), jnp.int32)
        one16i = jnp.full((16,), 1, jnp.int32)
        for c in range(nch + 1):
            cntbuf[pl.ds(c * 16, 16)] = zero16i

        def batch4(b4, _):
            gds = []
            for w in range(4):
                b = b4 * 4 + w
                for k in range(8):
                    sv = raw_s[pl.ds(b * 128 + k * 16, 16)]
                    dv = raw_d[pl.ds(b * 128 + k * 16, 16)]
                    cid = jnp.full((16,), nch, jnp.int32)
                    cnt_sel = cntbuf[pl.ds(nch * 16, 16)]
                    for c in range(nch):
                        mc = (dv >= c * CHUNK) & (dv < (c + 1) * CHUNK)
                        cv = cntbuf[pl.ds(c * 16, 16)]
                        cid = jnp.where(mc, jnp.full((16,), c, jnp.int32), cid)
                        cnt_sel = jnp.where(mc, cv, cnt_sel)
                        cntbuf[pl.ds(c * 16, 16)] = cv + jnp.where(
                            mc, one16i, zero16i)
                    mtr = dv >= n_dst
                    cntbuf[pl.ds(nch * 16, 16)] = (
                        cntbuf[pl.ds(nch * 16, 16)]
                        + jnp.where(mtr, one16i, zero16i))
                    dl = dv - cid * CHUNK
                    pos = (lane_base + cid) * capl + cnt_sel
                    svbuf[pl.ds(w * 128 + k * 16, 16)] = sv
                    dlbuf[pl.ds(w * 128 + k * 16, 16)] = dl
                    posbuf[w, pl.ds(k * 16, 16)] = pos
                gds.append(pltpu.async_copy(svbuf.at[pl.ds(w * 128, 128)],
                                            bks.at[posbuf.at[w]], semp))
                gds.append(pltpu.async_copy(dlbuf.at[pl.ds(w * 128, 128)],
                                            bkd.at[posbuf.at[w]], semp))
            for g in gds:
                g.wait()
            return 0

        lax.fori_loop(0, ept // 512, batch4, 0)
        dsv = (iota * 8 + t * 64) & 8191
        ddl = CHUNK + (iota & 63)
        for c in range(nch):
            cv = cntbuf[pl.ds(c * 16, 16)]
            for half in range(2):
                for k in range(8):
                    g = half * 8 + k
                    pos = (lane_base + c) * capl + cv + g
                    posbuf[0, pl.ds(k * 16, 16)] = pos
                    svbuf[pl.ds(k * 16, 16)] = dsv
                    dlbuf[pl.ds(k * 16, 16)] = ddl
                pltpu.sync_copy(svbuf.at[pl.ds(0, 128)], bks.at[posbuf.at[0]])
                pltpu.sync_copy(dlbuf.at[pl.ds(0, 128)], bkd.at[posbuf.at[0]])
        w = (nch + 1) * 16
        pltpu.sync_copy(cntbuf.at[pl.ds(0, w)], cnt.at[pl.ds(t * w, w)])

    @pl.when(sc == 0)
    def _():
        do_rel(sdd, ddd, bs_dd, bd_dd, cnt_dd, REL_DD, N_DRUG)
        do_rel(sdp, ddp, bs_dp, bd_dp, cnt_dp, REL_DP, N_PROT)

    @pl.when(sc == 1)
    def _():
        do_rel(spp, dpp, bs_pp, bd_pp, cnt_pp, REL_PP, N_PROT)
        do_rel(sde, dde, bs_de, bd_de, cnt_de, REL_DE, N_EFF)


def _prep(sdd, ddd, sdp, ddp, spp, dpp, sde, dde):
    mesh = plsc.VectorSubcoreMesh(core_axis_name="c", subcore_axis_name="s")
    ot = []
    for rel in (REL_DD, REL_DP, REL_PP, REL_DE):
        ot += [jax.ShapeDtypeStruct((_bk_len(rel),), jnp.int32),
               jax.ShapeDtypeStruct((_bk_len(rel),), jnp.int32),
               jax.ShapeDtypeStruct((_cnt_len(rel),), jnp.int32)]
    f = pl.kernel(
        _prep_body,
        out_type=ot,
        mesh=mesh,
        scratch_types=[
            pltpu.VMEM((EPT_MAX,), jnp.int32),
            pltpu.VMEM((EPT_MAX,), jnp.int32),
            pltpu.VMEM((512,), jnp.int32),
            pltpu.VMEM((512,), jnp.int32),
            pltpu.VMEM((4, 128), jnp.int32),
            pltpu.VMEM((256,), jnp.int32),
            pltpu.SemaphoreType.DMA,
        ],
    )
    return f(sdd, ddd, sdp, ddp, spp, dpp, sde, dde)


CBUF_G = EPT_MAX // 16 + 48


def _sums_body(xd, xp, bs_dd, bd_dd, cnt_dd, bs_dp, bd_dp, cnt_dp,
               bs_pp, bd_pp, cnt_pp, bs_de, bd_de, cnt_de,
               o_dd, o_dp, o_pp, o_de,
               bsbuf, bdbuf, cs, cd, cntvm, stg, sts, rows, zrow, acc,
               semg, semb):
    sc = lax.axis_index("c")
    t = lax.axis_index("s")
    iota = lax.iota(jnp.int32, 16)
    zero16 = jnp.zeros((16,), jnp.float32)
    dsv = (iota * 8 + t * 64) & 8191
    ddl = CHUNK + (iota & 63)

    def zinit(i, _):
        zrow[i // 8, pl.ds((i % 8) * 16, 16)] = zero16
        return 0
    lax.fori_loop(0, 512, zinit, 0)

    def zero_own_slice():
        r0 = t * CHUNK_PT
        for off, sz in DR_SPANS:
            pltpu.sync_copy(zrow.at[pl.ds(0, sz), :],
                            acc.at[pl.ds(r0 + off, sz), :])

    zero_own_slice()
    plsc.subcore_barrier()

    def do_rel(src_tab, bks, bkd, cnt, out, rel):
        ept, nch, k0 = rel["ept"], rel["nch"], rel["k0"]
        capl = ept // 16
        w = (nch + 1) * 16
        pltpu.sync_copy(cnt.at[pl.ds(t * w, w)], cntvm.at[pl.ds(0, w)])
        lo_ch = lax.select(sc == 0, 0, k0)
        hi_ch = lax.select(sc == 0, k0, nch)

        def chunk_body(ch, _):
            lo = ch * CHUNK
            bds = []
            for l in range(16):
                boff = ((t * 16 + l) * (nch + 1) + ch) * capl
                bds.append(pltpu.async_copy(
                    bks.at[pl.ds(boff, capl)],
                    bsbuf.at[pl.ds(l * capl, capl)], semb))
                bds.append(pltpu.async_copy(
                    bkd.at[pl.ds(boff, capl)],
                    bdbuf.at[pl.ds(l * capl, capl)], semb))
            for d in bds:
                d.wait()
            slot = 0
            for l in range(16):
                cl = cntvm[pl.ds(ch * 16 + l, 16)][0]
                ng = (cl + 15) // 16

                def g_loop(g, slot, l=l):
                    cs[pl.ds(slot * 16, 16)] = bsbuf[
                        pl.ds(l * capl + g * 16, 16)]
                    cd[pl.ds(slot * 16, 16)] = bdbuf[
                        pl.ds(l * capl + g * 16, 16)]
                    return slot + 1
                slot = lax.fori_loop(0, ng, g_loop, slot)
            for k in range(GB // 16):
                cs[pl.ds((slot + k) * 16, 16)] = dsv
                cd[pl.ds((slot + k) * 16, 16)] = ddl
            nb = (slot + GB // 16 - 1) // (GB // 16)

            def p2(b, _):
                for k in range(GB // 16):
                    r, col = k // 8, (k % 8) * 16
                    stg[r, pl.ds(col, 16)] = cs[pl.ds(b * GB + k * 16, 16)]
                    sts[r, pl.ds(col, 16)] = cd[pl.ds(b * GB + k * 16, 16)]
                gds = [pltpu.async_copy(src_tab.at[stg.at[r]],
                                        rows.at[pl.ds(r * 128, 128), :], semg)
                       for r in range(GB // 128)]
                for g in gds:
                    g.wait()
                for r in range(GB // 128):
                    pltpu.sync_copy(rows.at[pl.ds(r * 128, 128), :],
                                    acc.at[sts.at[r]], add=True)
                return 0
            lax.fori_loop(0, nb, p2, 0)
            plsc.subcore_barrier()
            r0 = t * CHUNK_PT
            for off, sz in DR_SPANS:
                pltpu.sync_copy(acc.at[pl.ds(r0 + off, sz), :],
                                out.at[pl.ds(lo + r0 + off, sz), :])
            zero_own_slice()
            plsc.subcore_barrier()
            return 0
        lax.fori_loop(lo_ch, hi_ch, chunk_body, 0)

    do_rel(xd, bs_dd, bd_dd, cnt_dd, o_dd, REL_DD)
    do_rel(xd, bs_dp, bd_dp, cnt_dp, o_dp, REL_DP)
    do_rel(xp, bs_pp, bd_pp, cnt_pp, o_pp, REL_PP)
    do_rel(xd, bs_de, bd_de, cnt_de, o_de, REL_DE)


def _sums(xd, xp, preps):
    mesh = plsc.VectorSubcoreMesh(core_axis_name="c", subcore_axis_name="s")
    f = pl.kernel(
        _sums_body,
        out_type=[jax.ShapeDtypeStruct((REL_DD["nch"] * CHUNK, H), jnp.float32),
                  jax.ShapeDtypeStruct((REL_DP["nch"] * CHUNK, H), jnp.float32),
                  jax.ShapeDtypeStruct((REL_PP["nch"] * CHUNK, H), jnp.float32),
                  jax.ShapeDtypeStruct((REL_DE["nch"] * CHUNK, H), jnp.float32)],
        mesh=mesh,
        scratch_types=[
            pltpu.VMEM((EPT_MAX,), jnp.int32),
            pltpu.VMEM((EPT_MAX,), jnp.int32),
            pltpu.VMEM((16 * CBUF_G,), jnp.int32),
            pltpu.VMEM((16 * CBUF_G,), jnp.int32),
            pltpu.VMEM((272,), jnp.int32),
            pltpu.VMEM((GB // 128, 128), jnp.int32),
            pltpu.VMEM((GB // 128, 128), jnp.int32),
            pltpu.VMEM((GB, H), jnp.float32),
            pltpu.VMEM((64, H), jnp.float32),
            pltpu.VMEM_SHARED((CHUNK + NDUMP, H), jnp.float32),
            pltpu.SemaphoreType.DMA,
            pltpu.SemaphoreType.DMA,
        ],
    )
    return f(xd, xp, *preps)


def _combine2_body(x_ref, s1_ref, r1_ref, s2_ref, r2_ref,
                   a1_ref, a2_ref, b_ref, c_ref, o_ref):
    x = x_ref[...]
    agg1 = s1_ref[...] * r1_ref[...]
    agg2 = s2_ref[...] * r2_ref[...]
    y = (x
         + jnp.dot(agg1, a1_ref[...], preferred_element_type=jnp.float32)
         + jnp.dot(agg2, a2_ref[...], preferred_element_type=jnp.float32)
         + jnp.dot(x, b_ref[...], preferred_element_type=jnp.float32)
         + c_ref[...])
    o_ref[...] = jnp.maximum(y, 0.0)


def _combine1_body(x_ref, s1_ref, r1_ref, a1_ref, b_ref, c_ref, o_ref):
    x = x_ref[...]
    agg1 = s1_ref[...] * r1_ref[...]
    y = (x
         + jnp.dot(agg1, a1_ref[...], preferred_element_type=jnp.float32)
         + jnp.dot(x, b_ref[...], preferred_element_type=jnp.float32)
         + c_ref[...])
    o_ref[...] = jnp.maximum(y, 0.0)


def _row_spec():
    return pl.BlockSpec((ROW_BLK, H), lambda i: (i, 0))


def _scalar_spec():
    return pl.BlockSpec((ROW_BLK, 1), lambda i: (i, 0))


def _full_spec(shape):
    return pl.BlockSpec(shape, lambda i: tuple(0 for _ in shape))


def _combine2(x, s1, r1, s2, r2, a1, a2, b, c):
    n = x.shape[0]
    grid = (pl.cdiv(n, ROW_BLK),)
    return pl.pallas_call(
        _combine2_body,
        grid=grid,
        in_specs=[_row_spec(), _row_spec(), _scalar_spec(), _row_spec(),
                  _scalar_spec(), _full_spec((H, H)), _full_spec((H, H)),
                  _full_spec((H, H)), _full_spec((1, H))],
        out_specs=_row_spec(),
        out_shape=jax.ShapeDtypeStruct((n, H), jnp.float32),
    )(x, s1, r1, s2, r2, a1, a2, b, c)


def _combine1(x, s1, r1, a1, b, c):
    n = x.shape[0]
    grid = (pl.cdiv(n, ROW_BLK),)
    return pl.pallas_call(
        _combine1_body,
        grid=grid,
        in_specs=[_row_spec(), _row_spec(), _scalar_spec(),
                  _full_spec((H, H)), _full_spec((H, H)), _full_spec((1, H))],
        out_specs=_row_spec(),
        out_shape=jax.ShapeDtypeStruct((n, H), jnp.float32),
    )(x, s1, r1, a1, b, c)


def _proj_body(x_ref, w1_ref, b1_ref, w2_ref, b2_ref, g_ref, beta_ref, o_ref):
    x = x_ref[...]
    h = jnp.maximum(
        jnp.dot(x, w1_ref[...], preferred_element_type=jnp.float32)
        + b1_ref[...], 0.0)
    h = jnp.dot(h, w2_ref[...], preferred_element_type=jnp.float32) + b2_ref[...]
    mu = jnp.mean(h, axis=-1, keepdims=True)
    var = jnp.mean((h - mu) ** 2, axis=-1, keepdims=True)
    o_ref[...] = (h - mu) * lax.rsqrt(var + 1e-5) * g_ref[...] + beta_ref[...]


def _proj(x, w1, b1, w2, b2, g, beta):
    n = x.shape[0]
    grid = (pl.cdiv(n, ROW_BLK),)
    return pl.pallas_call(
        _proj_body,
        grid=grid,
        in_specs=[_row_spec(), _full_spec((H, H)), _full_spec((1, H)),
                  _full_spec((H, H)), _full_spec((1, H)), _full_spec((1, H)),
                  _full_spec((1, H))],
        out_specs=_row_spec(),
        out_shape=jax.ShapeDtypeStruct((n, H), jnp.float32),
    )(x, w1, b1, w2, b2, g, beta)


def _pad_edges(ei, ept):
    e = ei.shape[1]
    total = 16 * ept
    src = jnp.pad(ei[0], (0, total - e))
    dst = jnp.pad(ei[1], (0, total - e), constant_values=SENTINEL)
    return src, dst


def kernel(emb_drug, emb_prot, emb_eff,
           Wl_dp, bl_dp, Wr_dp, Wl_dd, bl_dd, Wr_dd,
           Wl_pp, bl_pp, Wr_pp, Wl_de, bl_de, Wr_de,
           drug_W1, drug_b1, drug_W2, drug_b2, drug_g, drug_beta,
           prot_W1, prot_b1, prot_W2, prot_b2, prot_g, prot_beta,
           eff_W1, eff_b1, eff_W2, eff_b2, eff_g, eff_beta,
           ei_dp, ei_dd, ei_pp, ei_de):
    sdd, ddd = _pad_edges(ei_dd, EPT_DD)
    sdp, ddp = _pad_edges(ei_dp, EPT_DP)
    spp, dpp = _pad_edges(ei_pp, EPT_PP)
    sde, dde = _pad_edges(ei_de, EPT_DE)

    c_dd = jax.ops.segment_sum(jnp.ones((E_DD,), jnp.float32), ei_dd[1], num_segments=CL_DD)
    c_dp = jax.ops.segment_sum(jnp.ones((E_DP,), jnp.float32), ei_dp[1], num_segments=CL_DP)
    c_pp = jax.ops.segment_sum(jnp.ones((E_PP,), jnp.float32), ei_pp[1], num_segments=CL_DP)
    c_de = jax.ops.segment_sum(jnp.ones((E_DE,), jnp.float32), ei_de[1], num_segments=CL_DE)

    def recip(c, n):
        return (1.0 / jnp.maximum(c[:n].astype(jnp.float32), 1.0))[:, None]

    r_dd = recip(c_dd, N_DRUG)
    r_dp = recip(c_dp, N_PROT)
    r_pp = recip(c_pp, N_PROT)
    r_de = recip(c_de, N_EFF)

    preps = _prep(sdd, ddd, sdp, ddp, spp, dpp, sde, dde)

    def step(carry, ws):
        xd, xp, xe = carry
        (wldd, bldd, wrdd, wldp, bldp, wrdp,
         wlpp, blpp, wrpp, wlde, blde, wrde) = ws
        s_dd, s_dp, s_pp, s_de = _sums(xd, xp, preps)
        xd = _combine1(xd, s_dd, r_dd, wldd.T, wrdd.T, bldd[None, :])
        xp = _combine2(xp, s_dp, r_dp, s_pp, r_pp, wldp.T, wlpp.T,
                       (wrdp + wrpp).T, (bldp + blpp)[None, :])
        xe = _combine1(xe, s_de, r_de, wlde.T, wrde.T, blde[None, :])
        return (xd, xp, xe), None

    (xd, xp, xe), _ = lax.scan(
        step, (emb_drug, emb_prot, emb_eff),
        (Wl_dd, bl_dd, Wr_dd, Wl_dp, bl_dp, Wr_dp,
         Wl_pp, bl_pp, Wr_pp, Wl_de, bl_de, Wr_de))

    od = _proj(xd, drug_W1.T, drug_b1[None, :], drug_W2.T, drug_b2[None, :],
               drug_g[None, :], drug_beta[None, :])
    op = _proj(xp, prot_W1.T, prot_b1[None, :], prot_W2.T, prot_b2[None, :],
               prot_g[None, :], prot_beta[None, :])
    oe = _proj(xe, eff_W1.T, eff_b1[None, :], eff_W2.T, eff_b2[None, :],
               eff_g[None, :], eff_beta[None, :])
    return od, op, oe


# speedup vs baseline: 1.0007x; 1.0007x over previous
import functools

import jax
import jax.numpy as jnp
from jax import lax
from jax.experimental import pallas as pl
from jax.experimental.pallas import tpu as pltpu
from jax.experimental.pallas import tpu_sc as plsc

H = 128
L = 3
ROW_BLK = 512

N_DRUG, N_PROT, N_EFF = 10000, 50000, 20000
E_DP, E_DD, E_PP, E_DE = 200000, 100000, 200000, 100000

CHUNK = 3712
NDUMP = 64
GB = 256

SENTINEL = 2147483647


def _round_up(x, m):
    return (x + m - 1) // m * m


def _ept(e):
    return _round_up((e + 15) // 16, 512)


EPT_DP, EPT_DD, EPT_PP, EPT_DE = _ept(E_DP), _ept(E_DD), _ept(E_PP), _ept(E_DE)
EPT_MAX = max(EPT_DP, EPT_DD, EPT_PP, EPT_DE)

CP_DD = _round_up(N_DRUG // 16 + 1, 128)
CP_DP = _round_up(N_PROT // 16 + 1, 128)
CP_DE = _round_up(N_EFF // 16 + 1, 128)
CL_DD, CL_DP, CL_DE = 16 * CP_DD, 16 * CP_DP, 16 * CP_DE
CP_MAX = CP_DP
CL_MAX = CL_DP


SEG = CL_MAX
ZB = 1600


def _counts_body(ddd, ddp, dpp, dde,
                 o_dd, o_dp, o_pp, o_de,
                 raw_d, stg, ones_v, zbuf, csp):
    sc = lax.axis_index("c")
    t = lax.axis_index("s")
    iota = lax.iota(jnp.int32, 16)
    one16 = jnp.ones((16,), jnp.float32)
    zero16 = jnp.zeros((16,), jnp.float32)

    for k in range(8):
        ones_v[pl.ds(k * 16, 16)] = one16

    def z(i, _):
        zbuf[pl.ds(i * 16, 16)] = zero16
        return 0
    lax.fori_loop(0, ZB // 16, z, 0)
    for i in range(4):
        pltpu.sync_copy(zbuf.at[pl.ds(0, ZB)],
                        csp.at[pl.ds(t * 4 * ZB + i * ZB, ZB)])
    plsc.subcore_barrier()

    def do_rel(dsts, ept, n_dst, segbase):
        base = t * ept
        pltpu.sync_copy(dsts.at[pl.ds(base, ept)], raw_d.at[pl.ds(0, ept)])

        def b_loop(b, _):
            for k in range(8):
                dv = raw_d[pl.ds(b * 128 + k * 16, 16)]
                dv = jnp.where(dv < n_dst, dv, n_dst + iota) + segbase
                stg[0, pl.ds(k * 16, 16)] = dv
            pltpu.sync_copy(ones_v, csp.at[stg.at[0]], add=True)
            return 0
        lax.fori_loop(0, ept // 128, b_loop, 0)

    @pl.when(sc == 0)
    def _():
        do_rel(ddd, EPT_DD, N_DRUG, 0)
        do_rel(ddp, EPT_DP, N_PROT, SEG)

    @pl.when(sc == 1)
    def _():
        do_rel(dpp, EPT_PP, N_PROT, 0)
        do_rel(dde, EPT_DE, N_EFF, SEG)

    plsc.subcore_barrier()

    @pl.when(sc == 0)
    def _():
        pltpu.sync_copy(csp.at[pl.ds(t * CP_DD, CP_DD)],
                        o_dd.at[pl.ds(t * CP_DD, CP_DD)])
        pltpu.sync_copy(csp.at[pl.ds(SEG + t * CP_DP, CP_DP)],
                        o_dp.at[pl.ds(t * CP_DP, CP_DP)])

    @pl.when(sc == 1)
    def _():
        pltpu.sync_copy(csp.at[pl.ds(t * CP_DP, CP_DP)],
                        o_pp.at[pl.ds(t * CP_DP, CP_DP)])
        pltpu.sync_copy(csp.at[pl.ds(SEG + t * CP_DE, CP_DE)],
                        o_de.at[pl.ds(t * CP_DE, CP_DE)])


def _counts(ddd, ddp, dpp, dde):
    mesh = plsc.VectorSubcoreMesh(core_axis_name="c", subcore_axis_name="s")
    f = pl.kernel(
        _counts_body,
        out_type=[jax.ShapeDtypeStruct((CL_DD,), jnp.float32),
                  jax.ShapeDtypeStruct((CL_DP,), jnp.float32),
                  jax.ShapeDtypeStruct((CL_DP,), jnp.float32),
                  jax.ShapeDtypeStruct((CL_DE,), jnp.float32)],
        mesh=mesh,
        scratch_types=[
            pltpu.VMEM((EPT_MAX,), jnp.int32),
            pltpu.VMEM((1, 128), jnp.int32),
            pltpu.VMEM((128,), jnp.float32),
            pltpu.VMEM((ZB,), jnp.float32),
            pltpu.VMEM_SHARED((2 * CL_MAX,), jnp.float32),
        ],
    )
    return f(ddd, ddp, dpp, dde)


REL_DD = dict(nch=3, k0=2, ept=EPT_DD)
REL_DP = dict(nch=14, k0=7, ept=EPT_DP)
REL_PP = dict(nch=14, k0=7, ept=EPT_PP)
REL_DE = dict(nch=6, k0=3, ept=EPT_DE)

CHUNK_PT = CHUNK // 16
DR_SPANS = [(0, 64), (64, 64), (128, 64), (192, 40)]

SHIFT = 14
DMASK = (1 << SHIFT) - 1


def _bk_len(rel):
    return 256 * (rel["nch"] + 1) * (rel["ept"] // 16)


def _cnt_len(rel):
    return 16 * (rel["nch"] + 1) * 16


def _prep_body(sdd, ddd, sdp, ddp, spp, dpp, sde, dde,
               bs_dd, bd_dd, cnt_dd, bs_dp, bd_dp, cnt_dp,
               bs_pp, bd_pp, cnt_pp, bs_de, bd_de, cnt_de,
               raw_s, raw_d, svbuf, dlbuf, posbuf, cntbuf, semp):
    sc = lax.axis_index("c")
    t = lax.axis_index("s")
    iota = lax.iota(jnp.int32, 16)

    def do_rel(srcs, dsts, bks, bkd, cnt, rel, n_dst):
        ept, nch = rel["ept"], rel["nch"]
        capl = ept // 16
        base = t * ept
        pltpu.sync_copy(srcs.at[pl.ds(base, ept)], raw_s.at[pl.ds(0, ept)])
        pltpu.sync_copy(dsts.at[pl.ds(base, ept)], raw_d.at[pl.ds(0, ept)])
        lane_base = (t * 16 + iota) * (nch + 1)
        zero16i = jnp.zeros((16,), jnp.int32)
        one16i = jnp.full((16,), 1, jnp.int32)

        def batch4(b4, cnts):
            cnts = list(cnts)
            gds = []
            for w in range(4):
                b = b4 * 4 + w
                for k in range(8):
                    sv = raw_s[pl.ds(b * 128 + k * 16, 16)]
                    dv = raw_d[pl.ds(b * 128 + k * 16, 16)]
                    cid = jnp.full((16,), nch, jnp.int32)
                    cnt_sel = cnts[nch]
                    for c in range(nch):
                        mc = (dv >= c * CHUNK) & (dv < (c + 1) * CHUNK)
                        cid = jnp.where(mc, jnp.full((16,), c, jnp.int32), cid)
                        cnt_sel = jnp.where(mc, cnts[c], cnt_sel)
                        cnts[c] = cnts[c] + jnp.where(mc, one16i, zero16i)
                    mtr = dv >= n_dst
                    cnts[nch] = cnts[nch] + jnp.where(mtr, one16i, zero16i)
                    dl = dv - cid * CHUNK
                    pos = (lane_base + cid) * capl + cnt_sel
                    svbuf[pl.ds(w * 128 + k * 16, 16)] = sv
                    dlbuf[pl.ds(w * 128 + k * 16, 16)] = dl
                    posbuf[w, pl.ds(k * 16, 16)] = pos
                gds.append(pltpu.async_copy(svbuf.at[pl.ds(w * 128, 128)],
                                            bks.at[posbuf.at[w]], semp))
                gds.append(pltpu.async_copy(dlbuf.at[pl.ds(w * 128, 128)],
                                            bkd.at[posbuf.at[w]], semp))
            for g in gds:
                g.wait()
            return tuple(cnts)

        cnts = lax.fori_loop(0, ept // 512, batch4,
                             tuple(jnp.zeros((16,), jnp.int32)
                                   for _ in range(nch + 1)))
        for c in range(nch + 1):
            cntbuf[pl.ds(c * 16, 16)] = cnts[c]
        dsv = (iota * 8 + t * 64) & 8191
        ddl = CHUNK + (iota & 63)
        for c in range(nch):
            cv = cntbuf[pl.ds(c * 16, 16)]
            for half in range(2):
                for k in range(8):
                    g = half * 8 + k
                    pos = (lane_base + c) * capl + cv + g
                    posbuf[0, pl.ds(k * 16, 16)] = pos
                    svbuf[pl.ds(k * 16, 16)] = dsv
                    dlbuf[pl.ds(k * 16, 16)] = ddl
                pltpu.sync_copy(svbuf.at[pl.ds(0, 128)], bks.at[posbuf.at[0]])
                pltpu.sync_copy(dlbuf.at[pl.ds(0, 128)], bkd.at[posbuf.at[0]])
        w = (nch + 1) * 16
        pltpu.sync_copy(cntbuf.at[pl.ds(0, w)], cnt.at[pl.ds(t * w, w)])

    @pl.when(sc == 0)
    def _():
        do_rel(sdd, ddd, bs_dd, bd_dd, cnt_dd, REL_DD, N_DRUG)
        do_rel(sdp, ddp, bs_dp, bd_dp, cnt_dp, REL_DP, N_PROT)

    @pl.when(sc == 1)
    def _():
        do_rel(spp, dpp, bs_pp, bd_pp, cnt_pp, REL_PP, N_PROT)
        do_rel(sde, dde, bs_de, bd_de, cnt_de, REL_DE, N_EFF)


def _prep(sdd, ddd, sdp, ddp, spp, dpp, sde, dde):
    mesh = plsc.VectorSubcoreMesh(core_axis_name="c", subcore_axis_name="s")
    ot = []
    for rel in (REL_DD, REL_DP, REL_PP, REL_DE):
        ot += [jax.ShapeDtypeStruct((_bk_len(rel),), jnp.int32),
               jax.ShapeDtypeStruct((_bk_len(rel),), jnp.int32),
               jax.ShapeDtypeStruct((_cnt_len(rel),), jnp.int32)]
    f = pl.kernel(
        _prep_body,
        out_type=ot,
        mesh=mesh,
        scratch_types=[
            pltpu.VMEM((EPT_MAX,), jnp.int32),
            pltpu.VMEM((EPT_MAX,), jnp.int32),
            pltpu.VMEM((512,), jnp.int32),
            pltpu.VMEM((512,), jnp.int32),
            pltpu.VMEM((4, 128), jnp.int32),
            pltpu.VMEM((256,), jnp.int32),
            pltpu.SemaphoreType.DMA,
        ],
    )
    return f(sdd, ddd, sdp, ddp, spp, dpp, sde, dde)


CBUF_G = EPT_MAX // 16 + 48


def _sums_body(xd, xp, bs_dd, bd_dd, cnt_dd, bs_dp, bd_dp, cnt_dp,
               bs_pp, bd_pp, cnt_pp, bs_de, bd_de, cnt_de,
               o_dd, o_dp, o_pp, o_de,
               bsbuf, bdbuf, cs, cd, cntvm, stg, sts, rows, zrow, acc,
               semg, semb):
    sc = lax.axis_index("c")
    t = lax.axis_index("s")
    iota = lax.iota(jnp.int32, 16)
    zero16 = jnp.zeros((16,), jnp.float32)
    dsv = (iota * 8 + t * 64) & 8191
    ddl = CHUNK + (iota & 63)

    def zinit(i, _):
        zrow[i // 8, pl.ds((i % 8) * 16, 16)] = zero16
        return 0
    lax.fori_loop(0, 512, zinit, 0)

    def zero_own_slice():
        r0 = t * CHUNK_PT
        for off, sz in DR_SPANS:
            pltpu.sync_copy(zrow.at[pl.ds(0, sz), :],
                            acc.at[pl.ds(r0 + off, sz), :])

    zero_own_slice()
    plsc.subcore_barrier()

    def do_rel(src_tab, bks, bkd, cnt, out, rel):
        ept, nch, k0 = rel["ept"], rel["nch"], rel["k0"]
        capl = ept // 16
        w = (nch + 1) * 16
        pltpu.sync_copy(cnt.at[pl.ds(t * w, w)], cntvm.at[pl.ds(0, w)])
        lo_ch = lax.select(sc == 0, 0, k0)
        hi_ch = lax.select(sc == 0, k0, nch)

        def chunk_body(ch, _):
            lo = ch * CHUNK
            bds = []
            for l in range(16):
                boff = ((t * 16 + l) * (nch + 1) + ch) * capl
                bds.append(pltpu.async_copy(
                    bks.at[pl.ds(boff, capl)],
                    bsbuf.at[pl.ds(l * capl, capl)], semb))
                bds.append(pltpu.async_copy(
                    bkd.at[pl.ds(boff, capl)],
                    bdbuf.at[pl.ds(l * capl, capl)], semb))
            for d in bds:
                d.wait()
            slot = 0
            for l in range(16):
                cl = cntvm[pl.ds(ch * 16 + l, 16)][0]
                ng = (cl + 15) // 16

                def g_loop(g, slot, l=l):
                    cs[pl.ds(slot * 16, 16)] = bsbuf[
                        pl.ds(l * capl + g * 16, 16)]
                    cd[pl.ds(slot * 16, 16)] = bdbuf[
                        pl.ds(l * capl + g * 16, 16)]
                    return slot + 1
                slot = lax.fori_loop(0, ng, g_loop, slot)
            for k in range(GB // 16):
                cs[pl.ds((slot + k) * 16, 16)] = dsv
                cd[pl.ds((slot + k) * 16, 16)] = ddl
            nb = (slot + GB // 16 - 1) // (GB // 16)

            def p2(b, _):
                for k in range(GB // 16):
                    r, col = k // 8, (k % 8) * 16
                    stg[r, pl.ds(col, 16)] = cs[pl.ds(b * GB + k * 16, 16)]
                    sts[r, pl.ds(col, 16)] = cd[pl.ds(b * GB + k * 16, 16)]
                gds = [pltpu.async_copy(src_tab.at[stg.at[r]],
                                        rows.at[pl.ds(r * 128, 128), :], semg)
                       for r in range(GB // 128)]
                for g in gds:
                    g.wait()
                for r in range(GB // 128):
                    pltpu.sync_copy(rows.at[pl.ds(r * 128, 128), :],
                                    acc.at[sts.at[r]], add=True)
                return 0
            lax.fori_loop(0, nb, p2, 0)
            plsc.subcore_barrier()
            r0 = t * CHUNK_PT
            for off, sz in DR_SPANS:
                pltpu.sync_copy(acc.at[pl.ds(r0 + off, sz), :],
                                out.at[pl.ds(lo + r0 + off, sz), :])
            zero_own_slice()
            plsc.subcore_barrier()
            return 0
        lax.fori_loop(lo_ch, hi_ch, chunk_body, 0)

    do_rel(xd, bs_dd, bd_dd, cnt_dd, o_dd, REL_DD)
    do_rel(xd, bs_dp, bd_dp, cnt_dp, o_dp, REL_DP)
    do_rel(xp, bs_pp, bd_pp, cnt_pp, o_pp, REL_PP)
    do_rel(xd, bs_de, bd_de, cnt_de, o_de, REL_DE)


def _sums(xd, xp, preps):
    mesh = plsc.VectorSubcoreMesh(core_axis_name="c", subcore_axis_name="s")
    f = pl.kernel(
        _sums_body,
        out_type=[jax.ShapeDtypeStruct((REL_DD["nch"] * CHUNK, H), jnp.float32),
                  jax.ShapeDtypeStruct((REL_DP["nch"] * CHUNK, H), jnp.float32),
                  jax.ShapeDtypeStruct((REL_PP["nch"] * CHUNK, H), jnp.float32),
                  jax.ShapeDtypeStruct((REL_DE["nch"] * CHUNK, H), jnp.float32)],
        mesh=mesh,
        scratch_types=[
            pltpu.VMEM((EPT_MAX,), jnp.int32),
            pltpu.VMEM((EPT_MAX,), jnp.int32),
            pltpu.VMEM((16 * CBUF_G,), jnp.int32),
            pltpu.VMEM((16 * CBUF_G,), jnp.int32),
            pltpu.VMEM((272,), jnp.int32),
            pltpu.VMEM((GB // 128, 128), jnp.int32),
            pltpu.VMEM((GB // 128, 128), jnp.int32),
            pltpu.VMEM((GB, H), jnp.float32),
            pltpu.VMEM((64, H), jnp.float32),
            pltpu.VMEM_SHARED((CHUNK + NDUMP, H), jnp.float32),
            pltpu.SemaphoreType.DMA,
            pltpu.SemaphoreType.DMA,
        ],
    )
    return f(xd, xp, *preps)


def _combine2_body(x_ref, s1_ref, r1_ref, s2_ref, r2_ref,
                   a1_ref, a2_ref, b_ref, c_ref, o_ref):
    x = x_ref[...]
    agg1 = s1_ref[...] * r1_ref[...]
    agg2 = s2_ref[...] * r2_ref[...]
    y = (x
         + jnp.dot(agg1, a1_ref[...], preferred_element_type=jnp.float32)
         + jnp.dot(agg2, a2_ref[...], preferred_element_type=jnp.float32)
         + jnp.dot(x, b_ref[...], preferred_element_type=jnp.float32)
         + c_ref[...])
    o_ref[...] = jnp.maximum(y, 0.0)


def _combine1_body(x_ref, s1_ref, r1_ref, a1_ref, b_ref, c_ref, o_ref):
    x = x_ref[...]
    agg1 = s1_ref[...] * r1_ref[...]
    y = (x
         + jnp.dot(agg1, a1_ref[...], preferred_element_type=jnp.float32)
         + jnp.dot(x, b_ref[...], preferred_element_type=jnp.float32)
         + c_ref[...])
    o_ref[...] = jnp.maximum(y, 0.0)


def _row_spec():
    return pl.BlockSpec((ROW_BLK, H), lambda i: (i, 0))


def _scalar_spec():
    return pl.BlockSpec((ROW_BLK, 1), lambda i: (i, 0))


def _full_spec(shape):
    return pl.BlockSpec(shape, lambda i: tuple(0 for _ in shape))


def _combine2(x, s1, r1, s2, r2, a1, a2, b, c):
    n = x.shape[0]
    grid = (pl.cdiv(n, ROW_BLK),)
    return pl.pallas_call(
        _combine2_body,
        grid=grid,
        in_specs=[_row_spec(), _row_spec(), _scalar_spec(), _row_spec(),
                  _scalar_spec(), _full_spec((H, H)), _full_spec((H, H)),
                  _full_spec((H, H)), _full_spec((1, H))],
        out_specs=_row_spec(),
        out_shape=jax.ShapeDtypeStruct((n, H), jnp.float32),
    )(x, s1, r1, s2, r2, a1, a2, b, c)


def _combine1(x, s1, r1, a1, b, c):
    n = x.shape[0]
    grid = (pl.cdiv(n, ROW_BLK),)
    return pl.pallas_call(
        _combine1_body,
        grid=grid,
        in_specs=[_row_spec(), _row_spec(), _scalar_spec(),
                  _full_spec((H, H)), _full_spec((H, H)), _full_spec((1, H))],
        out_specs=_row_spec(),
        out_shape=jax.ShapeDtypeStruct((n, H), jnp.float32),
    )(x, s1, r1, a1, b, c)


def _proj_body(x_ref, w1_ref, b1_ref, w2_ref, b2_ref, g_ref, beta_ref, o_ref):
    x = x_ref[...]
    h = jnp.maximum(
        jnp.dot(x, w1_ref[...], preferred_element_type=jnp.float32)
        + b1_ref[...], 0.0)
    h = jnp.dot(h, w2_ref[...], preferred_element_type=jnp.float32) + b2_ref[...]
    mu = jnp.mean(h, axis=-1, keepdims=True)
    var = jnp.mean((h - mu) ** 2, axis=-1, keepdims=True)
    o_ref[...] = (h - mu) * lax.rsqrt(var + 1e-5) * g_ref[...] + beta_ref[...]


def _proj(x, w1, b1, w2, b2, g, beta):
    n = x.shape[0]
    grid = (pl.cdiv(n, ROW_BLK),)
    return pl.pallas_call(
        _proj_body,
        grid=grid,
        in_specs=[_row_spec(), _full_spec((H, H)), _full_spec((1, H)),
                  _full_spec((H, H)), _full_spec((1, H)), _full_spec((1, H)),
                  _full_spec((1, H))],
        out_specs=_row_spec(),
        out_shape=jax.ShapeDtypeStruct((n, H), jnp.float32),
    )(x, w1, b1, w2, b2, g, beta)


def _pad_edges(ei, ept):
    e = ei.shape[1]
    total = 16 * ept
    src = jnp.pad(ei[0], (0, total - e))
    dst = jnp.pad(ei[1], (0, total - e), constant_values=SENTINEL)
    return src, dst


def kernel(emb_drug, emb_prot, emb_eff,
           Wl_dp, bl_dp, Wr_dp, Wl_dd, bl_dd, Wr_dd,
           Wl_pp, bl_pp, Wr_pp, Wl_de, bl_de, Wr_de,
           drug_W1, drug_b1, drug_W2, drug_b2, drug_g, drug_beta,
           prot_W1, prot_b1, prot_W2, prot_b2, prot_g, prot_beta,
           eff_W1, eff_b1, eff_W2, eff_b2, eff_g, eff_beta,
           ei_dp, ei_dd, ei_pp, ei_de):
    sdd, ddd = _pad_edges(ei_dd, EPT_DD)
    sdp, ddp = _pad_edges(ei_dp, EPT_DP)
    spp, dpp = _pad_edges(ei_pp, EPT_PP)
    sde, dde = _pad_edges(ei_de, EPT_DE)

    c_dd = jax.ops.segment_sum(jnp.ones((E_DD,), jnp.float32), ei_dd[1], num_segments=CL_DD)
    c_dp = jax.ops.segment_sum(jnp.ones((E_DP,), jnp.float32), ei_dp[1], num_segments=CL_DP)
    c_pp = jax.ops.segment_sum(jnp.ones((E_PP,), jnp.float32), ei_pp[1], num_segments=CL_DP)
    c_de = jax.ops.segment_sum(jnp.ones((E_DE,), jnp.float32), ei_de[1], num_segments=CL_DE)

    def recip(c, n):
        return (1.0 / jnp.maximum(c[:n].astype(jnp.float32), 1.0))[:, None]

    r_dd = recip(c_dd, N_DRUG)
    r_dp = recip(c_dp, N_PROT)
    r_pp = recip(c_pp, N_PROT)
    r_de = recip(c_de, N_EFF)

    preps = _prep(sdd, ddd, sdp, ddp, spp, dpp, sde, dde)

    def step(carry, ws):
        xd, xp, xe = carry
        (wldd, bldd, wrdd, wldp, bldp, wrdp,
         wlpp, blpp, wrpp, wlde, blde, wrde) = ws
        s_dd, s_dp, s_pp, s_de = _sums(xd, xp, preps)
        xd = _combine1(xd, s_dd, r_dd, wldd.T, wrdd.T, bldd[None, :])
        xp = _combine2(xp, s_dp, r_dp, s_pp, r_pp, wldp.T, wlpp.T,
                       (wrdp + wrpp).T, (bldp + blpp)[None, :])
        xe = _combine1(xe, s_de, r_de, wlde.T, wrde.T, blde[None, :])
        return (xd, xp, xe), None

    (xd, xp, xe), _ = lax.scan(
        step, (emb_drug, emb_prot, emb_eff),
        (Wl_dd, bl_dd, Wr_dd, Wl_dp, bl_dp, Wr_dp,
         Wl_pp, bl_pp, Wr_pp, Wl_de, bl_de, Wr_de))

    od = _proj(xd, drug_W1.T, drug_b1[None, :], drug_W2.T, drug_b2[None, :],
               drug_g[None, :], drug_beta[None, :])
    op = _proj(xp, prot_W1.T, prot_b1[None, :], prot_W2.T, prot_b2[None, :],
               prot_g[None, :], prot_beta[None, :])
    oe = _proj(xe, eff_W1.T, eff_b1[None, :], eff_W2.T, eff_b2[None, :],
               eff_g[None, :], eff_beta[None, :])
    return od, op, oe
